# baseline (speedup 1.0000x reference)
import functools

import jax
import jax.numpy as jnp
from jax import lax
from jax.experimental import pallas as pl
from jax.experimental.pallas import tpu as pltpu
from jax.experimental.pallas import tpu_sc as plsc

VOCAB_ROWS = 1000000
EMB = 64
SCALE = 8.0
SPLIT = 512000

NC = 2
NS = 16
NW = NC * NS

BLK_B = 128
LSTEPS = 200
REPACK_C = 2048


def _repack_body(a_ref, b_ref, t2_ref):
    t2_ref[:, 0:EMB] = jnp.transpose(a_ref[...]) * SCALE
    t2_ref[:, EMB:2 * EMB] = jnp.transpose(b_ref[...]) * SCALE


def _repack(tT):
    nb = SPLIT // REPACK_C
    return pl.pallas_call(
        _repack_body,
        grid=(nb,),
        in_specs=[pl.BlockSpec((EMB, REPACK_C), lambda i: (0, i)),
                  pl.BlockSpec(
                      (EMB, REPACK_C),
                      lambda i: (0, jnp.minimum(
                          i + nb, VOCAB_ROWS // REPACK_C)))],
        out_specs=pl.BlockSpec((REPACK_C, 128), lambda i: (i, 0)),
        out_shape=jax.ShapeDtypeStruct((SPLIT, 128), jnp.float32),
    )(tT, tT)


def _gather_kernel():
    mesh = plsc.VectorSubcoreMesh(core_axis_name="c", subcore_axis_name="s")
    n_pairs = LSTEPS // 2

    @functools.partial(
        pl.kernel,
        mesh=mesh,
        out_type=jax.ShapeDtypeStruct((LSTEPS, EMB, NW * BLK_B), jnp.float32),
        compiler_params=pltpu.CompilerParams(
            use_tc_tiling_on_sc=True, needs_layout_passes=False),
        scratch_types=[
            pltpu.VMEM((LSTEPS, BLK_B), jnp.int32),
            pltpu.VMEM((BLK_B, 128), jnp.float32),
            pltpu.VMEM((BLK_B, 128), jnp.float32),
            pltpu.VMEM((BLK_B,), jnp.int32),
            pltpu.VMEM((BLK_B,), jnp.int32),
            pltpu.VMEM((BLK_B,), jnp.int32),
            pltpu.VMEM((BLK_B,), jnp.int32),
            pltpu.VMEM((EMB, BLK_B), jnp.float32),
            pltpu.VMEM((EMB, BLK_B), jnp.float32),
            pltpu.SemaphoreType.DMA,
            pltpu.SemaphoreType.DMA,
            pltpu.SemaphoreType.DMA,
            pltpu.SemaphoreType.DMA,
        ],
    )
    def body(tokT_hbm, t2_hbm, out_hbm, tokv, g0, g1, r0, r1, h0, h1,
             p0, p1, gs0, gs1, os0, os1):
        c = lax.axis_index("c")
        s = lax.axis_index("s")
        wid = s * NC + c
        b0 = wid * BLK_B

        pltpu.sync_copy(tokT_hbm.at[:, pl.ds(b0, BLK_B)], tokv)

        def compute_idx(l, r_ref, h_ref):
            for k in range(0, BLK_B, 16):
                t = tokv[l, pl.ds(k, 16)]
                ge = t >= SPLIT
                r_ref[pl.ds(k, 16)] = t - jnp.where(ge, SPLIT, 0)
                h_ref[pl.ds(k, 16)] = jnp.where(ge, EMB, 0)

        def fire_g(r_ref, g_ref, sem):
            pltpu.async_copy(t2_hbm.at[r_ref], g_ref, sem)

        def wait_g(g_ref, sem):
            pltpu.make_async_copy(
                t2_hbm.at[pl.ds(0, BLK_B)], g_ref, sem).wait()

        def fire_o(l, p_ref, sem):
            pltpu.async_copy(
                p_ref, out_hbm.at[l, :, pl.ds(b0, BLK_B)], sem)

        def wait_o(p_ref, sem):
            pltpu.make_async_copy(
                p_ref, out_hbm.at[0, :, pl.ds(b0, BLK_B)], sem).wait()

        def transpose(g_ref, h_ref, p_ref):
            for k in range(0, BLK_B, 16):
                kvec = lax.iota(jnp.int32, 16) + k
                hv = h_ref[pl.ds(k, 16)]

                def erow(e, carry):
                    vals = plsc.load_gather(g_ref, [kvec, hv + e])
                    p_ref[e, pl.ds(k, 16)] = vals
                    return carry

                lax.fori_loop(0, EMB, erow, 0, unroll=4)

        compute_idx(0, r0, h0)
        fire_g(r0, g0, gs0)

        def pair(si, carry):
            l0 = 2 * si

            compute_idx(l0 + 1, r1, h1)
            fire_g(r1, g1, gs1)

            wait_g(g0, gs0)

            @pl.when(si > 0)
            def _():
                wait_o(p0, os0)
            transpose(g0, h0, p0)
            fire_o(l0, p0, os0)

            @pl.when(si < n_pairs - 1)
            def _():
                compute_idx(l0 + 2, r0, h0)
                fire_g(r0, g0, gs0)

            wait_g(g1, gs1)

            @pl.when(si > 0)
            def _():
                wait_o(p1, os1)
            transpose(g1, h1, p1)
            fire_o(l0 + 1, p1, os1)
            return carry

        lax.fori_loop(0, n_pairs, pair, 0)
        wait_o(p0, os0)
        wait_o(p1, os1)

    return body


@jax.jit
def _lookup(tokens, table):
    t2 = _repack(table.T)
    tokT = tokens.T.astype(jnp.int32)
    out3 = _gather_kernel()(tokT, t2)
    return jnp.transpose(out3, (2, 0, 1))


def kernel(tokens, table):
    b, l = tokens.shape
    out = _lookup(tokens, table)
    return out.reshape(b, l, EMB)


# speedup vs baseline: 1.5663x; 1.5663x over previous
import functools

import jax
import jax.numpy as jnp
from jax import lax
from jax.experimental import pallas as pl
from jax.experimental.pallas import tpu as pltpu
from jax.experimental.pallas import tpu_sc as plsc

VOCAB_ROWS = 1000000
EMB = 64
SCALE = 8.0
SPLIT = 512000

NC = 2
NS = 16
NW = NC * NS

BLK_B = 128
LSTEPS = 200
REPACK_C = 2048


def _repack_body(a_ref, b_ref, t2_ref):
    t2_ref[:, 0:EMB] = jnp.transpose(a_ref[...]) * SCALE
    t2_ref[:, EMB:2 * EMB] = jnp.transpose(b_ref[...]) * SCALE


def _repack(tT):
    nb = SPLIT // REPACK_C
    return pl.pallas_call(
        _repack_body,
        grid=(nb,),
        in_specs=[pl.BlockSpec((EMB, REPACK_C), lambda i: (0, i)),
                  pl.BlockSpec(
                      (EMB, REPACK_C),
                      lambda i: (0, jnp.minimum(
                          i + nb, VOCAB_ROWS // REPACK_C)))],
        out_specs=pl.BlockSpec((REPACK_C, 128), lambda i: (i, 0)),
        out_shape=jax.ShapeDtypeStruct((SPLIT, 128), jnp.float32),
    )(tT, tT)


def _gather_kernel():
    mesh = plsc.VectorSubcoreMesh(core_axis_name="c", subcore_axis_name="s")
    n_pairs = LSTEPS // 2

    @functools.partial(
        pl.kernel,
        mesh=mesh,
        out_type=jax.ShapeDtypeStruct((LSTEPS, EMB, NW * BLK_B), jnp.float32),
        compiler_params=pltpu.CompilerParams(
            use_tc_tiling_on_sc=True, needs_layout_passes=False),
        scratch_types=[
            pltpu.VMEM((LSTEPS, BLK_B), jnp.int32),
            pltpu.VMEM((BLK_B, 128), jnp.float32),
            pltpu.VMEM((BLK_B, 128), jnp.float32),
            pltpu.VMEM((BLK_B,), jnp.int32),
            pltpu.VMEM((BLK_B,), jnp.int32),
            pltpu.VMEM((BLK_B,), jnp.int32),
            pltpu.VMEM((BLK_B,), jnp.int32),
            pltpu.VMEM((EMB, BLK_B), jnp.float32),
            pltpu.VMEM((EMB, BLK_B), jnp.float32),
            pltpu.SemaphoreType.DMA,
            pltpu.SemaphoreType.DMA,
            pltpu.SemaphoreType.DMA,
            pltpu.SemaphoreType.DMA,
        ],
    )
    def body(tokT_hbm, t2_hbm, out_hbm, tokv, g0, g1, r0, r1, h0, h1,
             p0, p1, gs0, gs1, os0, os1):
        c = lax.axis_index("c")
        s = lax.axis_index("s")
        wid = s * NC + c
        b0 = wid * BLK_B

        pltpu.sync_copy(tokT_hbm.at[:, pl.ds(b0, BLK_B)], tokv)

        def compute_idx(l, r_ref, h_ref):
            for k in range(0, BLK_B, 16):
                t = tokv[l, pl.ds(k, 16)]
                ge = t >= SPLIT
                r_ref[pl.ds(k, 16)] = t - jnp.where(ge, SPLIT, 0)
                h_ref[pl.ds(k, 16)] = jnp.where(ge, EMB, 0)

        def fire_g(r_ref, g_ref, sem):
            pltpu.async_copy(t2_hbm.at[r_ref], g_ref, sem)

        def wait_g(g_ref, sem):
            pltpu.make_async_copy(
                t2_hbm.at[pl.ds(0, BLK_B)], g_ref, sem).wait()

        def fire_o(l, p_ref, sem):
            pltpu.async_copy(
                p_ref, out_hbm.at[l, :, pl.ds(b0, BLK_B)], sem)

        def wait_o(p_ref, sem):
            pltpu.make_async_copy(
                p_ref, out_hbm.at[0, :, pl.ds(b0, BLK_B)], sem).wait()

        def transpose(g_ref, h_ref, p_ref):
            for k in range(0, BLK_B, 16):
                kvec = lax.iota(jnp.int32, 16) + k
                hv = h_ref[pl.ds(k, 16)]

                @plsc.parallel_loop(0, EMB, unroll=8)
                def _(e):
                    vals = plsc.load_gather(g_ref, [kvec, hv + e])
                    p_ref[e, pl.ds(k, 16)] = vals

        compute_idx(0, r0, h0)
        fire_g(r0, g0, gs0)

        def pair(si, carry):
            l0 = 2 * si

            compute_idx(l0 + 1, r1, h1)
            fire_g(r1, g1, gs1)

            wait_g(g0, gs0)

            @pl.when(si > 0)
            def _():
                wait_o(p0, os0)
            transpose(g0, h0, p0)
            fire_o(l0, p0, os0)

            @pl.when(si < n_pairs - 1)
            def _():
                compute_idx(l0 + 2, r0, h0)
                fire_g(r0, g0, gs0)

            wait_g(g1, gs1)

            @pl.when(si > 0)
            def _():
                wait_o(p1, os1)
            transpose(g1, h1, p1)
            fire_o(l0 + 1, p1, os1)
            return carry

        lax.fori_loop(0, n_pairs, pair, 0)
        wait_o(p0, os0)
        wait_o(p1, os1)

    return body


@jax.jit
def _lookup(tokens, table):
    t2 = _repack(table.T)
    tokT = tokens.T.astype(jnp.int32)
    out3 = _gather_kernel()(tokT, t2)
    return jnp.transpose(out3, (2, 0, 1))


def kernel(tokens, table):
    b, l = tokens.shape
    out = _lookup(tokens, table)
    return out.reshape(b, l, EMB)


# speedup vs baseline: 2.6672x; 1.7029x over previous
import functools

import jax
import jax.numpy as jnp
from jax import lax
from jax.experimental import pallas as pl
from jax.experimental.pallas import tpu as pltpu
from jax.experimental.pallas import tpu_sc as plsc

VOCAB_ROWS = 1000000
EMB = 64
SCALE = 8.0
SPLIT = 512000

NC = 2
NS = 16
NW = NC * NS

BLK_B = 128
LSTEPS = 200
REPACK_C = 2048


def _repack_body(a_ref, b_ref, t2_ref):
    t2_ref[:, 0:EMB] = jnp.transpose(a_ref[...]) * SCALE
    t2_ref[:, EMB:2 * EMB] = jnp.transpose(b_ref[...]) * SCALE


def _repack(tT):
    nb = SPLIT // REPACK_C
    return pl.pallas_call(
        _repack_body,
        grid=(nb,),
        in_specs=[pl.BlockSpec((EMB, REPACK_C), lambda i: (0, i)),
                  pl.BlockSpec(
                      (EMB, REPACK_C),
                      lambda i: (0, jnp.minimum(
                          i + nb, VOCAB_ROWS // REPACK_C)))],
        out_specs=pl.BlockSpec((REPACK_C, 128), lambda i: (i, 0)),
        out_shape=jax.ShapeDtypeStruct((SPLIT, 128), jnp.float32),
    )(tT, tT)


def _gather_kernel():
    mesh = plsc.VectorSubcoreMesh(core_axis_name="c", subcore_axis_name="s")
    n_pairs = LSTEPS // 2

    @functools.partial(
        pl.kernel,
        mesh=mesh,
        out_type=jax.ShapeDtypeStruct((LSTEPS, EMB, NW * BLK_B), jnp.float32),
        compiler_params=pltpu.CompilerParams(
            use_tc_tiling_on_sc=True, needs_layout_passes=False),
        scratch_types=[
            pltpu.VMEM((LSTEPS, BLK_B), jnp.int32),
            pltpu.VMEM((BLK_B, 128), jnp.float32),
            pltpu.VMEM((BLK_B, 128), jnp.float32),
            pltpu.VMEM((BLK_B,), jnp.int32),
            pltpu.VMEM((BLK_B,), jnp.int32),
            pltpu.VMEM((BLK_B,), jnp.int32),
            pltpu.VMEM((BLK_B,), jnp.int32),
            pltpu.VMEM((EMB, BLK_B), jnp.float32),
            pltpu.VMEM((EMB, BLK_B), jnp.float32),
            pltpu.SemaphoreType.DMA,
            pltpu.SemaphoreType.DMA,
            pltpu.SemaphoreType.DMA,
            pltpu.SemaphoreType.DMA,
        ],
    )
    def body(tokT_hbm, t2_hbm, out_hbm, tokv, g0, g1, r0, r1, h0, h1,
             p0, p1, gs0, gs1, os0, os1):
        c = lax.axis_index("c")
        s = lax.axis_index("s")
        wid = s * NC + c
        b0 = wid * BLK_B

        pltpu.sync_copy(tokT_hbm.at[:, pl.ds(b0, BLK_B)], tokv)

        def compute_idx(l, r_ref, h_ref):
            for k in range(0, BLK_B, 16):
                t = tokv[l, pl.ds(k, 16)]
                ge = t >= SPLIT
                r_ref[pl.ds(k, 16)] = t - jnp.where(ge, SPLIT, 0)
                h_ref[pl.ds(k, 16)] = jnp.where(ge, EMB, 0)

        def fire_g(r_ref, g_ref, sem):
            pltpu.async_copy(t2_hbm.at[r_ref], g_ref, sem)

        def wait_g(g_ref, sem):
            pltpu.make_async_copy(
                t2_hbm.at[pl.ds(0, BLK_B)], g_ref, sem).wait()

        def fire_o(l, p_ref, sem):
            pltpu.async_copy(
                p_ref, out_hbm.at[l, :, pl.ds(b0, BLK_B)], sem)

        def wait_o(p_ref, sem):
            pltpu.make_async_copy(
                p_ref, out_hbm.at[0, :, pl.ds(b0, BLK_B)], sem).wait()

        def transpose(g_ref, h_ref, p_ref):
            for k in range(0, BLK_B, 16):
                kvec = lax.iota(jnp.int32, 16) + k
                hv = h_ref[pl.ds(k, 16)]

                @plsc.parallel_loop(0, EMB, unroll=8)
                def _(e):
                    r = (e + kvec) & (EMB - 1)
                    vals = plsc.load_gather(g_ref, [kvec, hv + r])
                    plsc.store_scatter(p_ref, [r, kvec], vals)

        compute_idx(0, r0, h0)
        fire_g(r0, g0, gs0)

        def pair(si, carry):
            l0 = 2 * si

            compute_idx(l0 + 1, r1, h1)
            fire_g(r1, g1, gs1)

            wait_g(g0, gs0)

            @pl.when(si > 0)
            def _():
                wait_o(p0, os0)
            transpose(g0, h0, p0)
            fire_o(l0, p0, os0)

            @pl.when(si < n_pairs - 1)
            def _():
                compute_idx(l0 + 2, r0, h0)
                fire_g(r0, g0, gs0)

            wait_g(g1, gs1)

            @pl.when(si > 0)
            def _():
                wait_o(p1, os1)
            transpose(g1, h1, p1)
            fire_o(l0 + 1, p1, os1)
            return carry

        lax.fori_loop(0, n_pairs, pair, 0)
        wait_o(p0, os0)
        wait_o(p1, os1)

    return body


@jax.jit
def _lookup(tokens, table):
    t2 = _repack(table.T)
    tokT = tokens.T.astype(jnp.int32)
    out3 = _gather_kernel()(tokT, t2)
    return jnp.transpose(out3, (2, 0, 1))


def kernel(tokens, table):
    b, l = tokens.shape
    out = _lookup(tokens, table)
    return out.reshape(b, l, EMB)


# speedup vs baseline: 3.0126x; 1.1295x over previous
import functools

import jax
import jax.numpy as jnp
from jax import lax
from jax.experimental import pallas as pl
from jax.experimental.pallas import tpu as pltpu
from jax.experimental.pallas import tpu_sc as plsc

VOCAB_ROWS = 1000000
EMB = 64
SCALE = 8.0
SPLIT = 512000

NC = 2
NS = 16
NW = NC * NS

BLK_B = 128
LSTEPS = 200
REPACK_C = 4096


def _repack_body(a_ref, b_ref, t2_ref):
    t2_ref[:, 0:EMB] = jnp.transpose(a_ref[...]) * SCALE
    t2_ref[:, EMB:2 * EMB] = jnp.transpose(b_ref[...]) * SCALE


def _repack(tT):
    nb = SPLIT // REPACK_C
    return pl.pallas_call(
        _repack_body,
        grid=(nb,),
        in_specs=[pl.BlockSpec((EMB, REPACK_C), lambda i: (0, i)),
                  pl.BlockSpec(
                      (EMB, REPACK_C),
                      lambda i: (0, jnp.minimum(
                          i + nb, VOCAB_ROWS // REPACK_C)))],
        out_specs=pl.BlockSpec((REPACK_C, 128), lambda i: (i, 0)),
        out_shape=jax.ShapeDtypeStruct((SPLIT, 128), jnp.float32),
    )(tT, tT)


def _gather_kernel():
    mesh = plsc.VectorSubcoreMesh(core_axis_name="c", subcore_axis_name="s")
    n_pairs = LSTEPS // 2

    @functools.partial(
        pl.kernel,
        mesh=mesh,
        out_type=jax.ShapeDtypeStruct((LSTEPS, EMB, NW * BLK_B), jnp.float32),
        compiler_params=pltpu.CompilerParams(
            use_tc_tiling_on_sc=True, needs_layout_passes=False),
        scratch_types=[
            pltpu.VMEM((LSTEPS, BLK_B), jnp.int32),
            pltpu.VMEM((BLK_B, 128), jnp.float32),
            pltpu.VMEM((BLK_B, 128), jnp.float32),
            pltpu.VMEM((BLK_B,), jnp.int32),
            pltpu.VMEM((BLK_B,), jnp.int32),
            pltpu.VMEM((BLK_B,), jnp.int32),
            pltpu.VMEM((BLK_B,), jnp.int32),
            pltpu.VMEM((EMB, BLK_B), jnp.float32),
            pltpu.VMEM((EMB, BLK_B), jnp.float32),
            pltpu.SemaphoreType.DMA,
            pltpu.SemaphoreType.DMA,
            pltpu.SemaphoreType.DMA,
            pltpu.SemaphoreType.DMA,
        ],
    )
    def body(tokT_hbm, t2_hbm, out_hbm, tokv, g0, g1, r0, r1, h0, h1,
             p0, p1, gs0, gs1, os0, os1):
        c = lax.axis_index("c")
        s = lax.axis_index("s")
        wid = s * NC + c
        b0 = wid * BLK_B

        pltpu.sync_copy(tokT_hbm.at[:, pl.ds(b0, BLK_B)], tokv)

        def compute_idx(l, r_ref, h_ref):
            for k in range(0, BLK_B, 16):
                t = tokv[l, pl.ds(k, 16)]
                ge = t >= SPLIT
                r_ref[pl.ds(k, 16)] = t - jnp.where(ge, SPLIT, 0)
                h_ref[pl.ds(k, 16)] = jnp.where(ge, EMB, 0)

        def fire_g(r_ref, g_ref, sem):
            pltpu.async_copy(t2_hbm.at[r_ref], g_ref, sem)

        def wait_g(g_ref, sem):
            pltpu.make_async_copy(
                t2_hbm.at[pl.ds(0, BLK_B)], g_ref, sem).wait()

        def fire_o(l, p_ref, sem):
            pltpu.async_copy(
                p_ref, out_hbm.at[l, :, pl.ds(b0, BLK_B)], sem)

        def wait_o(p_ref, sem):
            pltpu.make_async_copy(
                p_ref, out_hbm.at[0, :, pl.ds(b0, BLK_B)], sem).wait()

        def transpose(g_ref, h_ref, p_ref):
            for k in range(0, BLK_B, 16):
                kvec = lax.iota(jnp.int32, 16) + k
                hv = h_ref[pl.ds(k, 16)]

                @plsc.parallel_loop(0, EMB, unroll=8)
                def _(e):
                    r = (e + kvec) & (EMB - 1)
                    vals = plsc.load_gather(g_ref, [kvec, hv + r])
                    plsc.store_scatter(p_ref, [r, kvec], vals)

        compute_idx(0, r0, h0)
        fire_g(r0, g0, gs0)

        def pair(si, carry):
            l0 = 2 * si

            compute_idx(l0 + 1, r1, h1)
            fire_g(r1, g1, gs1)

            wait_g(g0, gs0)

            @pl.when(si > 0)
            def _():
                wait_o(p0, os0)
            transpose(g0, h0, p0)
            fire_o(l0, p0, os0)

            @pl.when(si < n_pairs - 1)
            def _():
                compute_idx(l0 + 2, r0, h0)
                fire_g(r0, g0, gs0)

            wait_g(g1, gs1)

            @pl.when(si > 0)
            def _():
                wait_o(p1, os1)
            transpose(g1, h1, p1)
            fire_o(l0 + 1, p1, os1)
            return carry

        lax.fori_loop(0, n_pairs, pair, 0)
        wait_o(p0, os0)
        wait_o(p1, os1)

    return body


@jax.jit
def _lookup(tokens, table):
    t2 = _repack(table.T)
    tokT = tokens.T.astype(jnp.int32)
    out3 = _gather_kernel()(tokT, t2)
    return jnp.transpose(out3, (2, 0, 1))


def kernel(tokens, table):
    b, l = tokens.shape
    out = _lookup(tokens, table)
    return out.reshape(b, l, EMB)


# speedup vs baseline: 3.2607x; 1.0824x over previous
import functools

import jax
import jax.numpy as jnp
from jax import lax
from jax.experimental import pallas as pl
from jax.experimental.pallas import tpu as pltpu
from jax.experimental.pallas import tpu_sc as plsc

VOCAB_ROWS = 1000000
EMB = 64
SCALE = 8.0
SPLIT = 512000

NC = 2
NS = 16
NW = NC * NS

BLK_B = 128
LSTEPS = 200
REPACK_C = 10240


def _repack_body(a_ref, b_ref, t2_ref):
    t2_ref[:, 0:EMB] = jnp.transpose(a_ref[...]) * SCALE
    t2_ref[:, EMB:2 * EMB] = jnp.transpose(b_ref[...]) * SCALE


def _repack(tT):
    nb = SPLIT // REPACK_C
    return pl.pallas_call(
        _repack_body,
        grid=(nb,),
        in_specs=[pl.BlockSpec((EMB, REPACK_C), lambda i: (0, i)),
                  pl.BlockSpec(
                      (EMB, REPACK_C),
                      lambda i: (0, jnp.minimum(
                          i + nb, VOCAB_ROWS // REPACK_C)))],
        out_specs=pl.BlockSpec((REPACK_C, 128), lambda i: (i, 0)),
        out_shape=jax.ShapeDtypeStruct((SPLIT, 128), jnp.float32),
    )(tT, tT)


def _gather_kernel():
    mesh = plsc.VectorSubcoreMesh(core_axis_name="c", subcore_axis_name="s")
    n_pairs = LSTEPS // 2

    @functools.partial(
        pl.kernel,
        mesh=mesh,
        out_type=jax.ShapeDtypeStruct((LSTEPS, EMB, NW * BLK_B), jnp.float32),
        compiler_params=pltpu.CompilerParams(
            use_tc_tiling_on_sc=True, needs_layout_passes=False),
        scratch_types=[
            pltpu.VMEM((LSTEPS, BLK_B), jnp.int32),
            pltpu.VMEM((BLK_B, 128), jnp.float32),
            pltpu.VMEM((BLK_B, 128), jnp.float32),
            pltpu.VMEM((BLK_B,), jnp.int32),
            pltpu.VMEM((BLK_B,), jnp.int32),
            pltpu.VMEM((BLK_B,), jnp.int32),
            pltpu.VMEM((BLK_B,), jnp.int32),
            pltpu.VMEM((EMB, BLK_B), jnp.float32),
            pltpu.VMEM((EMB, BLK_B), jnp.float32),
            pltpu.SemaphoreType.DMA,
            pltpu.SemaphoreType.DMA,
            pltpu.SemaphoreType.DMA,
            pltpu.SemaphoreType.DMA,
        ],
    )
    def body(tokT_hbm, t2_hbm, out_hbm, tokv, g0, g1, r0, r1, h0, h1,
             p0, p1, gs0, gs1, os0, os1):
        c = lax.axis_index("c")
        s = lax.axis_index("s")
        wid = s * NC + c
        b0 = wid * BLK_B

        pltpu.sync_copy(tokT_hbm.at[:, pl.ds(b0, BLK_B)], tokv)

        def compute_idx(l, r_ref, h_ref):
            for k in range(0, BLK_B, 16):
                t = tokv[l, pl.ds(k, 16)]
                ge = t >= SPLIT
                r_ref[pl.ds(k, 16)] = t - jnp.where(ge, SPLIT, 0)
                h_ref[pl.ds(k, 16)] = jnp.where(ge, EMB, 0)

        def fire_g(r_ref, g_ref, sem):
            pltpu.async_copy(t2_hbm.at[r_ref], g_ref, sem)

        def wait_g(g_ref, sem):
            pltpu.make_async_copy(
                t2_hbm.at[pl.ds(0, BLK_B)], g_ref, sem).wait()

        def fire_o(l, p_ref, sem):
            pltpu.async_copy(
                p_ref, out_hbm.at[l, :, pl.ds(b0, BLK_B)], sem)

        def wait_o(p_ref, sem):
            pltpu.make_async_copy(
                p_ref, out_hbm.at[0, :, pl.ds(b0, BLK_B)], sem).wait()

        def transpose(g_ref, h_ref, p_ref):
            for k in range(0, BLK_B, 16):
                kvec = lax.iota(jnp.int32, 16) + k
                hv = h_ref[pl.ds(k, 16)]

                @plsc.parallel_loop(0, EMB, unroll=8)
                def _(e):
                    r = (e + kvec) & (EMB - 1)
                    vals = plsc.load_gather(g_ref, [kvec, hv + r])
                    plsc.store_scatter(p_ref, [r, kvec], vals)

        compute_idx(0, r0, h0)
        fire_g(r0, g0, gs0)

        def pair(si, carry):
            l0 = 2 * si

            compute_idx(l0 + 1, r1, h1)
            fire_g(r1, g1, gs1)

            wait_g(g0, gs0)

            @pl.when(si > 0)
            def _():
                wait_o(p0, os0)
            transpose(g0, h0, p0)
            fire_o(l0, p0, os0)

            @pl.when(si < n_pairs - 1)
            def _():
                compute_idx(l0 + 2, r0, h0)
                fire_g(r0, g0, gs0)

            wait_g(g1, gs1)

            @pl.when(si > 0)
            def _():
                wait_o(p1, os1)
            transpose(g1, h1, p1)
            fire_o(l0 + 1, p1, os1)
            return carry

        lax.fori_loop(0, n_pairs, pair, 0)
        wait_o(p0, os0)
        wait_o(p1, os1)

    return body


@jax.jit
def _lookup(tokens, table):
    t2 = _repack(table.T)
    tokT = tokens.T.astype(jnp.int32)
    out3 = _gather_kernel()(tokT, t2)
    return jnp.transpose(out3, (2, 0, 1))


def kernel(tokens, table):
    b, l = tokens.shape
    out = _lookup(tokens, table)
    return out.reshape(b, l, EMB)


# speedup vs baseline: 3.3040x; 1.0133x over previous
import functools

import jax
import jax.numpy as jnp
from jax import lax
from jax.experimental import pallas as pl
from jax.experimental.pallas import tpu as pltpu
from jax.experimental.pallas import tpu_sc as plsc

VOCAB_ROWS = 1000000
EMB = 64
SCALE = 8.0
SPLIT = 512000

NC = 2
NS = 16
NW = NC * NS

BLK_B = 128
LSTEPS = 200
REPACK_C = 20480


def _repack_body(a_ref, b_ref, t2_ref):
    t2_ref[:, 0:EMB] = jnp.transpose(a_ref[...]) * SCALE
    t2_ref[:, EMB:2 * EMB] = jnp.transpose(b_ref[...]) * SCALE


def _repack(tT):
    nb = SPLIT // REPACK_C
    return pl.pallas_call(
        _repack_body,
        grid=(nb,),
        in_specs=[pl.BlockSpec((EMB, REPACK_C), lambda i: (0, i)),
                  pl.BlockSpec(
                      (EMB, REPACK_C),
                      lambda i: (0, jnp.minimum(
                          i + nb, VOCAB_ROWS // REPACK_C)))],
        out_specs=pl.BlockSpec((REPACK_C, 128), lambda i: (i, 0)),
        out_shape=jax.ShapeDtypeStruct((SPLIT, 128), jnp.float32),
    )(tT, tT)


def _gather_kernel():
    mesh = plsc.VectorSubcoreMesh(core_axis_name="c", subcore_axis_name="s")
    n_pairs = LSTEPS // 2

    @functools.partial(
        pl.kernel,
        mesh=mesh,
        out_type=jax.ShapeDtypeStruct((LSTEPS, EMB, NW * BLK_B), jnp.float32),
        compiler_params=pltpu.CompilerParams(
            use_tc_tiling_on_sc=True, needs_layout_passes=False),
        scratch_types=[
            pltpu.VMEM((LSTEPS, BLK_B), jnp.int32),
            pltpu.VMEM((BLK_B, 128), jnp.float32),
            pltpu.VMEM((BLK_B, 128), jnp.float32),
            pltpu.VMEM((BLK_B,), jnp.int32),
            pltpu.VMEM((BLK_B,), jnp.int32),
            pltpu.VMEM((BLK_B,), jnp.int32),
            pltpu.VMEM((BLK_B,), jnp.int32),
            pltpu.VMEM((EMB, BLK_B), jnp.float32),
            pltpu.VMEM((EMB, BLK_B), jnp.float32),
            pltpu.SemaphoreType.DMA,
            pltpu.SemaphoreType.DMA,
            pltpu.SemaphoreType.DMA,
            pltpu.SemaphoreType.DMA,
        ],
    )
    def body(tokT_hbm, t2_hbm, out_hbm, tokv, g0, g1, r0, r1, h0, h1,
             p0, p1, gs0, gs1, os0, os1):
        c = lax.axis_index("c")
        s = lax.axis_index("s")
        wid = s * NC + c
        b0 = wid * BLK_B

        pltpu.sync_copy(tokT_hbm.at[:, pl.ds(b0, BLK_B)], tokv)

        def compute_idx(l, r_ref, h_ref):
            for k in range(0, BLK_B, 16):
                t = tokv[l, pl.ds(k, 16)]
                ge = t >= SPLIT
                r_ref[pl.ds(k, 16)] = t - jnp.where(ge, SPLIT, 0)
                h_ref[pl.ds(k, 16)] = jnp.where(ge, EMB, 0)

        def fire_g(r_ref, g_ref, sem):
            pltpu.async_copy(t2_hbm.at[r_ref], g_ref, sem)

        def wait_g(g_ref, sem):
            pltpu.make_async_copy(
                t2_hbm.at[pl.ds(0, BLK_B)], g_ref, sem).wait()

        def fire_o(l, p_ref, sem):
            pltpu.async_copy(
                p_ref, out_hbm.at[l, :, pl.ds(b0, BLK_B)], sem)

        def wait_o(p_ref, sem):
            pltpu.make_async_copy(
                p_ref, out_hbm.at[0, :, pl.ds(b0, BLK_B)], sem).wait()

        def transpose(g_ref, h_ref, p_ref):
            for k in range(0, BLK_B, 16):
                kvec = lax.iota(jnp.int32, 16) + k
                hv = h_ref[pl.ds(k, 16)]

                @plsc.parallel_loop(0, EMB, unroll=8)
                def _(e):
                    r = (e + kvec) & (EMB - 1)
                    vals = plsc.load_gather(g_ref, [kvec, hv + r])
                    plsc.store_scatter(p_ref, [r, kvec], vals)

        compute_idx(0, r0, h0)
        fire_g(r0, g0, gs0)

        def pair(si, carry):
            l0 = 2 * si

            compute_idx(l0 + 1, r1, h1)
            fire_g(r1, g1, gs1)

            wait_g(g0, gs0)

            @pl.when(si > 0)
            def _():
                wait_o(p0, os0)
            transpose(g0, h0, p0)
            fire_o(l0, p0, os0)

            @pl.when(si < n_pairs - 1)
            def _():
                compute_idx(l0 + 2, r0, h0)
                fire_g(r0, g0, gs0)

            wait_g(g1, gs1)

            @pl.when(si > 0)
            def _():
                wait_o(p1, os1)
            transpose(g1, h1, p1)
            fire_o(l0 + 1, p1, os1)
            return carry

        lax.fori_loop(0, n_pairs, pair, 0)
        wait_o(p0, os0)
        wait_o(p1, os1)

    return body


@jax.jit
def _lookup(tokens, table):
    t2 = _repack(table.T)
    tokT = tokens.T.astype(jnp.int32)
    out3 = _gather_kernel()(tokT, t2)
    return jnp.transpose(out3, (2, 0, 1))


def kernel(tokens, table):
    b, l = tokens.shape
    out = _lookup(tokens, table)
    return out.reshape(b, l, EMB)
